# Initial kernel scaffold; baseline (speedup 1.0000x reference)
import jax
import jax.numpy as jnp
from jax.experimental import pallas as pl


def kernel(token_ids, hidden, emb, wih, whh, bih, bhh):
    raise NotImplementedError("write your pallas kernel here")


# speedup vs baseline: 1.6501x; 1.6501x over previous
import functools

import jax
import jax.numpy as jnp
from jax import lax
from jax.experimental import pallas as pl
from jax.experimental.pallas import tpu as pltpu


def _gru_fused_kernel(tok_ids_ref,
                      emb_ref,
                      h0_ref,
                      wih_ref, whh_ref,
                      bgi_ref,
                      bhn_ref,
                      out_ref,
                      gi_ref,
                      *, hidden_size, seq_len, chunk):
    H = hidden_size
    T = seq_len
    C = chunk

    wih = wih_ref[...]
    bgi = bgi_ref[...]

    def proj_body(i, carry):
        base = i * C
        rows = [emb_ref[tok_ids_ref[base + k]] for k in range(C)]
        x = jnp.concatenate(rows, axis=0)
        gi_ref[pl.ds(base, C), :] = (
            jnp.dot(x, wih, preferred_element_type=jnp.float32) + bgi)
        return carry

    lax.fori_loop(0, T // C, proj_body, 0)

    whh = whh_ref[...]
    bhn = bhn_ref[...]

    def rec_body(i, h):
        base = i * C
        gi_c = gi_ref[pl.ds(base, C), :]
        for k in range(C):
            gh = jnp.dot(h, whh, preferred_element_type=jnp.float32)
            gi = gi_c[k:k + 1, :]
            r = jax.nn.sigmoid(gi[:, :H] + gh[:, :H])
            z = jax.nn.sigmoid(gi[:, H:2 * H] + gh[:, H:2 * H])
            n = jnp.tanh(gi[:, 2 * H:] + r * (gh[:, 2 * H:] + bhn))
            h = (1.0 - z) * n + z * h
            out_ref[base + k] = h
        return h

    lax.fori_loop(0, T // C, rec_body, h0_ref[...])


@jax.jit
def kernel(token_ids, hidden, emb, wih, whh, bih, bhh):
    T = token_ids.shape[0]
    V, _, H = emb.shape
    C = 8

    h0 = hidden.reshape(1, H).astype(jnp.float32)
    zeros_n = jnp.zeros((1, H), dtype=jnp.float32)
    bgi = bih + jnp.concatenate([bhh[:, :2 * H], zeros_n], axis=1)
    bhn = bhh[:, 2 * H:]

    body = functools.partial(_gru_fused_kernel, hidden_size=H, seq_len=T,
                             chunk=C)

    outs = pl.pallas_call(
        body,
        out_shape=jax.ShapeDtypeStruct((T, 1, H), hidden.dtype),
        grid_spec=pltpu.PrefetchScalarGridSpec(
            num_scalar_prefetch=1,
            grid=(1,),
            in_specs=[
                pl.BlockSpec((V, 1, H), lambda i, toks: (0, 0, 0)),
                pl.BlockSpec((1, H), lambda i, toks: (0, 0)),
                pl.BlockSpec((H, 3 * H), lambda i, toks: (0, 0)),
                pl.BlockSpec((H, 3 * H), lambda i, toks: (0, 0)),
                pl.BlockSpec((1, 3 * H), lambda i, toks: (0, 0)),
                pl.BlockSpec((1, H), lambda i, toks: (0, 0)),
            ],
            out_specs=pl.BlockSpec((T, 1, H), lambda i, toks: (0, 0, 0)),
            scratch_shapes=[pltpu.VMEM((T, 3 * H), jnp.float32)],
        ),
        compiler_params=pltpu.CompilerParams(
            dimension_semantics=("arbitrary",)),
    )(token_ids.astype(jnp.int32), emb, h0, wih, whh, bgi, bhn)

    final_hidden = outs[T - 1].reshape(1, 1, H)
    return outs, final_hidden


# speedup vs baseline: 1.8388x; 1.1143x over previous
import functools

import jax
import jax.numpy as jnp
from jax import lax
from jax.experimental import pallas as pl
from jax.experimental.pallas import tpu as pltpu


def _gru_fused_kernel(tok_ids_ref,
                      emb_ref,
                      h0_ref,
                      wih_ref, whh_ref,
                      bgi_ref,
                      bhn_ref,
                      out_ref,
                      x_ref,
                      gi_ref,
                      *, hidden_size, seq_len, chunk):
    H = hidden_size
    T = seq_len
    C = chunk

    def gather_body(i, carry):
        base = i * C
        for k in range(C):
            x_ref[base + k] = emb_ref[tok_ids_ref[base + k]]
        return carry

    lax.fori_loop(0, T // C, gather_body, 0)

    x = x_ref[...].reshape(T, H)
    gi_ref[...] = (
        jnp.dot(x, wih_ref[...], preferred_element_type=jnp.float32)
        + bgi_ref[...])

    whh = whh_ref[...]
    bhn = bhn_ref[...]

    def rec_body(i, h):
        base = i * C
        gi_c = gi_ref[pl.ds(base, C), :]
        for k in range(C):
            gh = jnp.dot(h, whh, preferred_element_type=jnp.float32)
            gi = gi_c[k:k + 1, :]
            r = 0.5 * (jnp.tanh(0.5 * (gi[:, :H] + gh[:, :H])) + 1.0)
            z = 0.5 * (jnp.tanh(0.5 * (gi[:, H:2 * H] + gh[:, H:2 * H])) + 1.0)
            n = jnp.tanh(gi[:, 2 * H:] + r * (gh[:, 2 * H:] + bhn))
            h = (1.0 - z) * n + z * h
            out_ref[base + k] = h
        return h

    lax.fori_loop(0, T // C, rec_body, h0_ref[...])


@jax.jit
def kernel(token_ids, hidden, emb, wih, whh, bih, bhh):
    T = token_ids.shape[0]
    V, _, H = emb.shape
    C = 8

    h0 = hidden.reshape(1, H).astype(jnp.float32)
    zeros_n = jnp.zeros((1, H), dtype=jnp.float32)
    bgi = bih + jnp.concatenate([bhh[:, :2 * H], zeros_n], axis=1)
    bhn = bhh[:, 2 * H:]

    body = functools.partial(_gru_fused_kernel, hidden_size=H, seq_len=T,
                             chunk=C)

    outs = pl.pallas_call(
        body,
        out_shape=jax.ShapeDtypeStruct((T, 1, H), hidden.dtype),
        grid_spec=pltpu.PrefetchScalarGridSpec(
            num_scalar_prefetch=1,
            grid=(1,),
            in_specs=[
                pl.BlockSpec((V, 1, H), lambda i, toks: (0, 0, 0)),
                pl.BlockSpec((1, H), lambda i, toks: (0, 0)),
                pl.BlockSpec((H, 3 * H), lambda i, toks: (0, 0)),
                pl.BlockSpec((H, 3 * H), lambda i, toks: (0, 0)),
                pl.BlockSpec((1, 3 * H), lambda i, toks: (0, 0)),
                pl.BlockSpec((1, H), lambda i, toks: (0, 0)),
            ],
            out_specs=pl.BlockSpec((T, 1, H), lambda i, toks: (0, 0, 0)),
            scratch_shapes=[pltpu.VMEM((T, 1, H), jnp.float32),
                            pltpu.VMEM((T, 3 * H), jnp.float32)],
        ),
        compiler_params=pltpu.CompilerParams(
            dimension_semantics=("arbitrary",)),
    )(token_ids.astype(jnp.int32), emb, h0, wih, whh, bgi, bhn)

    final_hidden = outs[T - 1].reshape(1, 1, H)
    return outs, final_hidden


# speedup vs baseline: 2.0165x; 1.0966x over previous
import functools

import jax
import jax.numpy as jnp
from jax import lax
from jax.experimental import pallas as pl
from jax.experimental.pallas import tpu as pltpu


def _gru_fused_kernel(tok_ids_ref,
                      emb_ref,
                      hid_ref,
                      wih_ref,
                      whh_ref,
                      bih_ref, bhh_ref,
                      out_ref,
                      fin_ref,
                      x_ref,
                      gi_ref,
                      *, hidden_size, seq_len, chunk):
    H = hidden_size
    T = seq_len
    C = chunk

    def gather_body(i, carry):
        base = i * C
        for k in range(C):
            x_ref[base + k] = emb_ref[tok_ids_ref[base + k]]
        return carry

    lax.fori_loop(0, T // C, gather_body, 0)

    lane = lax.broadcasted_iota(jnp.int32, (1, 3 * H), 1)
    bias = bih_ref[...] + jnp.where(lane < 2 * H, bhh_ref[...], 0.0)

    x = x_ref[...].reshape(T, H)
    gi_ref[...] = (
        jnp.dot(x, wih_ref[...], preferred_element_type=jnp.float32) + bias)

    bhn = bhh_ref[:, 2 * H:]

    def rec_body(i, h):
        base = i * C
        gi_c = gi_ref[pl.ds(base, C), :]
        for k in range(C):
            gh = jnp.dot(h.astype(jnp.bfloat16), whh_ref[...],
                         preferred_element_type=jnp.float32)
            gi = gi_c[k:k + 1, :]
            r = 0.5 * (jnp.tanh(0.5 * (gi[:, :H] + gh[:, :H])) + 1.0)
            z = 0.5 * (jnp.tanh(0.5 * (gi[:, H:2 * H] + gh[:, H:2 * H])) + 1.0)
            n = jnp.tanh(gi[:, 2 * H:] + r * (gh[:, 2 * H:] + bhn))
            h = (1.0 - z) * n + z * h
            out_ref[base + k] = h
        return h

    h_final = lax.fori_loop(0, T // C, rec_body, hid_ref[0])
    fin_ref[0] = h_final


@jax.jit
def kernel(token_ids, hidden, emb, wih, whh, bih, bhh):
    T = token_ids.shape[0]
    V, _, H = emb.shape
    C = 8

    body = functools.partial(_gru_fused_kernel, hidden_size=H, seq_len=T,
                             chunk=C)

    outs, fin = pl.pallas_call(
        body,
        out_shape=[jax.ShapeDtypeStruct((T, 1, H), hidden.dtype),
                   jax.ShapeDtypeStruct((1, 1, H), hidden.dtype)],
        grid_spec=pltpu.PrefetchScalarGridSpec(
            num_scalar_prefetch=1,
            grid=(1,),
            in_specs=[
                pl.BlockSpec((V, 1, H), lambda i, toks: (0, 0, 0)),
                pl.BlockSpec((1, 1, H), lambda i, toks: (0, 0, 0)),
                pl.BlockSpec((H, 3 * H), lambda i, toks: (0, 0)),
                pl.BlockSpec((H, 3 * H), lambda i, toks: (0, 0)),
                pl.BlockSpec((1, 3 * H), lambda i, toks: (0, 0)),
                pl.BlockSpec((1, 3 * H), lambda i, toks: (0, 0)),
            ],
            out_specs=[pl.BlockSpec((T, 1, H), lambda i, toks: (0, 0, 0)),
                       pl.BlockSpec((1, 1, H), lambda i, toks: (0, 0, 0))],
            scratch_shapes=[pltpu.VMEM((T, 1, H), jnp.float32),
                            pltpu.VMEM((T, 3 * H), jnp.float32)],
        ),
        compiler_params=pltpu.CompilerParams(
            dimension_semantics=("arbitrary",)),
    )(token_ids.astype(jnp.int32), emb, hidden, wih,
      whh.astype(jnp.bfloat16), bih, bhh)

    return outs, fin


# speedup vs baseline: 2.2774x; 1.1294x over previous
import functools

import jax
import jax.numpy as jnp
from jax import lax
from jax.experimental import pallas as pl
from jax.experimental.pallas import tpu as pltpu


def _gru_fused_kernel(tok_ids_ref,
                      emb_ref,
                      hid_ref,
                      wih_ref,
                      whh_ref,
                      bih_ref, bhh_ref,
                      out_ref,
                      fin_ref,
                      x_ref,
                      gi_ref,
                      wrz_ref,
                      wn_ref,
                      *, hidden_size, seq_len, chunk):
    H = hidden_size
    T = seq_len
    C = chunk

    def gather_body(i, carry):
        base = i * C
        for k in range(C):
            x_ref[base + k] = emb_ref[tok_ids_ref[base + k]]
        return carry

    lax.fori_loop(0, T // C, gather_body, 0)

    lane = lax.broadcasted_iota(jnp.int32, (1, 3 * H), 1)
    bias = bih_ref[...] + jnp.where(lane < 2 * H, bhh_ref[...], 0.0)

    x = x_ref[...].reshape(T, H)
    gi_ref[...] = (
        jnp.dot(x, wih_ref[...], preferred_element_type=jnp.float32) + bias)

    bhn = bhh_ref[:, 2 * H:]
    wrz_ref[...] = whh_ref[:, :2 * H].astype(jnp.bfloat16)
    wn_ref[...] = whh_ref[:, 2 * H:].astype(jnp.bfloat16)

    def rec_body(i, h):
        base = i * C
        gi_c = gi_ref[pl.ds(base, C), :]
        for k in range(C):
            gh = jnp.dot(h.astype(jnp.bfloat16), wrz_ref[...],
                         preferred_element_type=jnp.float32)
            ghn = jnp.sum(h.reshape(H, 1) * wn_ref[...], axis=0,
                          keepdims=True)
            gi = gi_c[k:k + 1, :]
            r = 0.5 * (jnp.tanh(0.5 * (gi[:, :H] + gh[:, :H])) + 1.0)
            z = 0.5 * (jnp.tanh(0.5 * (gi[:, H:2 * H] + gh[:, H:])) + 1.0)
            n = jnp.tanh(gi[:, 2 * H:] + r * (ghn + bhn))
            h = (1.0 - z) * n + z * h
            out_ref[base + k] = h
        return h

    h_final = lax.fori_loop(0, T // C, rec_body, hid_ref[0])
    fin_ref[0] = h_final


@jax.jit
def kernel(token_ids, hidden, emb, wih, whh, bih, bhh):
    T = token_ids.shape[0]
    V, _, H = emb.shape
    C = 16

    body = functools.partial(_gru_fused_kernel, hidden_size=H, seq_len=T,
                             chunk=C)

    outs, fin = pl.pallas_call(
        body,
        out_shape=[jax.ShapeDtypeStruct((T, 1, H), hidden.dtype),
                   jax.ShapeDtypeStruct((1, 1, H), hidden.dtype)],
        grid_spec=pltpu.PrefetchScalarGridSpec(
            num_scalar_prefetch=1,
            grid=(1,),
            in_specs=[
                pl.BlockSpec((V, 1, H), lambda i, toks: (0, 0, 0)),
                pl.BlockSpec((1, 1, H), lambda i, toks: (0, 0, 0)),
                pl.BlockSpec((H, 3 * H), lambda i, toks: (0, 0)),
                pl.BlockSpec((H, 3 * H), lambda i, toks: (0, 0)),
                pl.BlockSpec((1, 3 * H), lambda i, toks: (0, 0)),
                pl.BlockSpec((1, 3 * H), lambda i, toks: (0, 0)),
            ],
            out_specs=[pl.BlockSpec((T, 1, H), lambda i, toks: (0, 0, 0)),
                       pl.BlockSpec((1, 1, H), lambda i, toks: (0, 0, 0))],
            scratch_shapes=[pltpu.VMEM((T, 1, H), jnp.float32),
                            pltpu.VMEM((T, 3 * H), jnp.float32),
                            pltpu.VMEM((H, 2 * H), jnp.bfloat16),
                            pltpu.VMEM((H, H), jnp.bfloat16)],
        ),
        compiler_params=pltpu.CompilerParams(
            dimension_semantics=("arbitrary",)),
    )(token_ids.astype(jnp.int32), emb, hidden, wih, whh, bih, bhh)

    return outs, fin
